# baseline (speedup 1.0000x reference)
import functools

import jax
import jax.numpy as jnp
from jax import lax
from jax.experimental import pallas as pl
from jax.experimental.pallas import tpu as pltpu
from jax.experimental.pallas import tpu_sc as plsc

_D = 64
_L = 128
_V = 1000000
_NW = 32


def _pack_table(table_t):
    blk = 1024
    grid = -(-_V // blk)

    def body(x_ref, o_ref):
        i1 = jax.lax.broadcasted_iota(jnp.int32, (_D, _D), 0)
        i2 = jax.lax.broadcasted_iota(jnp.int32, (_D, _D), 1)
        eye = (i1 == i2).astype(jnp.float32)
        o_ref[:, : _D] = jax.lax.dot_general(
            x_ref[...], eye, (((0,), (0,)), ((), ())),
            preferred_element_type=jnp.float32)

    return pl.pallas_call(
        body,
        grid=(grid,),
        in_specs=[pl.BlockSpec((_D, blk), lambda g: (0, g))],
        out_specs=pl.BlockSpec((blk, _L), lambda g: (g, 0)),
        out_shape=jax.ShapeDtypeStruct((_V, _L), jnp.float32),
    )(table_t)


def _sc_gather(scratch, ids_t):
    n_s, n_b = ids_t.shape
    b_groups = n_b // _L
    tasks_pw = n_s * b_groups // _NW
    supers_pw = tasks_pw // 8

    mesh = plsc.VectorSubcoreMesh(core_axis_name="c", subcore_axis_name="s")

    @functools.partial(
        pl.kernel,
        out_type=jax.ShapeDtypeStruct((n_s * n_b, _L), jnp.float32),
        mesh=mesh,
        scratch_types=[
            pltpu.VMEM((tasks_pw, _L), jnp.int32),
            pltpu.VMEM((_L, _L), jnp.float32),
            pltpu.VMEM((_L, _L), jnp.float32),
            pltpu.SemaphoreType.DMA,
            pltpu.SemaphoreType.DMA,
            pltpu.SemaphoreType.DMA,
            pltpu.SemaphoreType.DMA,
            pltpu.SemaphoreType.DMA,
        ],
        compiler_params=pltpu.CompilerParams(
            use_tc_tiling_on_sc=True, needs_layout_passes=False),
    )
    def k(scr, ids, out, ix_all, r0, r1, g0, g1, xs, o0, o1):
        wid = lax.axis_index("s") * 2 + lax.axis_index("c")
        rows = (r0, r1)
        gsem = (g0, g1)
        osem = (o0, o1)

        def stage(j, carry):
            g8 = wid * supers_pw + j
            s8 = g8 // b_groups
            b128 = g8 % b_groups
            pltpu.async_copy(
                ids.at[pl.ds(s8 * 8, 8), pl.ds(b128 * _L, _L)],
                ix_all.at[pl.ds(j * 8, 8)], xs)
            return carry

        lax.fori_loop(0, supers_pw, stage, 0)
        pltpu.make_async_copy(
            ids.at[pl.ds(0, tasks_pw), pl.ds(0, _L)], ix_all, xs).wait()

        def rowbase(t):
            g8 = wid * supers_pw + t // 8
            s = (g8 // b_groups) * 8 + t % 8
            return s * n_b + (g8 % b_groups) * _L

        def start_g(t, b):
            pltpu.async_copy(scr.at[ix_all.at[t]], rows[b], gsem[b])

        def wait_g(b):
            pltpu.make_async_copy(
                scr.at[pl.ds(0, _L)], rows[b], gsem[b]).wait()

        def start_o(t, b):
            pltpu.async_copy(
                rows[b], out.at[pl.ds(rowbase(t), _L)], osem[b])

        def wait_o(b):
            pltpu.make_async_copy(
                out.at[pl.ds(0, _L)], rows[b], osem[b]).wait()

        start_g(0, 0)
        start_g(1, 1)
        wait_g(0)
        start_o(0, 0)

        def body(i, carry):
            t_a = 2 * i + 1
            wait_o(0)
            start_g(t_a + 1, 0)
            wait_g(1)
            start_o(t_a, 1)
            wait_o(1)
            start_g(t_a + 2, 1)
            wait_g(0)
            start_o(t_a + 1, 0)
            return carry

        lax.fori_loop(0, (tasks_pw - 2) // 2, body, 0)
        wait_g(1)
        start_o(tasks_pw - 1, 1)
        wait_o(0)
        wait_o(1)

    return k(scratch, ids_t)


def _unpack_out(out128):
    n_s, n_b, blk = 200, 4096, 512

    def body(x_ref, o_ref):
        i1 = jax.lax.broadcasted_iota(jnp.int32, (_L, _L), 0)
        i2 = jax.lax.broadcasted_iota(jnp.int32, (_L, _L), 1)
        eye = (i1 == i2).astype(jnp.float32)
        y = jax.lax.dot_general(
            eye, x_ref[...], (((0,), (1,)), ((), ())),
            preferred_element_type=jnp.float32)
        o_ref[...] = y[: _D].reshape(1, _D, blk)

    return pl.pallas_call(
        body,
        grid=(n_s, n_b // blk),
        in_specs=[pl.BlockSpec((blk, _L), lambda s, b: (s * (n_b // blk) + b, 0))],
        out_specs=pl.BlockSpec((1, _D, blk), lambda s, b: (s, 0, b)),
        out_shape=jax.ShapeDtypeStruct((n_s, _D, n_b), jnp.float32),
    )(out128)


def kernel(input_ids, embedding_table):
    table_t = jnp.transpose(embedding_table)
    ids_t = jnp.transpose(input_ids.astype(jnp.int32))
    scratch = _pack_table(table_t)
    out128 = _sc_gather(scratch, ids_t)
    out3 = _unpack_out(out128)
    return jnp.transpose(out3, (2, 0, 1))


# speedup vs baseline: 2.0485x; 2.0485x over previous
import functools

import jax
import jax.numpy as jnp
from jax import lax
from jax.experimental import pallas as pl
from jax.experimental.pallas import tpu as pltpu
from jax.experimental.pallas import tpu_sc as plsc

_D = 64
_L = 128
_V = 1000000
_NW = 32


def _pack_table(table_t):
    blk = 4096
    grid = -(-_V // blk)

    def body(x_ref, o_ref):
        o_ref[:, : _D] = jnp.transpose(x_ref[...])

    return pl.pallas_call(
        body,
        grid=(grid,),
        in_specs=[pl.BlockSpec((_D, blk), lambda g: (0, g))],
        out_specs=pl.BlockSpec((blk, _L), lambda g: (g, 0)),
        out_shape=jax.ShapeDtypeStruct((_V, _L), jnp.float32),
    )(table_t)


def _sc_gather(scratch, ids_t):
    n_s, n_b = ids_t.shape
    b_groups = n_b // _L
    tasks_pw = n_s * b_groups // _NW
    supers_pw = tasks_pw // 8

    mesh = plsc.VectorSubcoreMesh(core_axis_name="c", subcore_axis_name="s")

    @functools.partial(
        pl.kernel,
        out_type=jax.ShapeDtypeStruct((n_s * n_b, _L), jnp.float32),
        mesh=mesh,
        scratch_types=[
            pltpu.VMEM((tasks_pw, _L), jnp.int32),
            pltpu.VMEM((_L, _L), jnp.float32),
            pltpu.VMEM((_L, _L), jnp.float32),
            pltpu.SemaphoreType.DMA,
            pltpu.SemaphoreType.DMA,
            pltpu.SemaphoreType.DMA,
            pltpu.SemaphoreType.DMA,
            pltpu.SemaphoreType.DMA,
        ],
        compiler_params=pltpu.CompilerParams(
            use_tc_tiling_on_sc=True, needs_layout_passes=False),
    )
    def k(scr, ids, out, ix_all, r0, r1, g0, g1, xs, o0, o1):
        wid = lax.axis_index("s") * 2 + lax.axis_index("c")
        rows = (r0, r1)
        gsem = (g0, g1)
        osem = (o0, o1)

        def stage(j, carry):
            g8 = wid * supers_pw + j
            s8 = g8 // b_groups
            b128 = g8 % b_groups
            pltpu.async_copy(
                ids.at[pl.ds(s8 * 8, 8), pl.ds(b128 * _L, _L)],
                ix_all.at[pl.ds(j * 8, 8)], xs)
            return carry

        lax.fori_loop(0, supers_pw, stage, 0)
        pltpu.make_async_copy(
            ids.at[pl.ds(0, tasks_pw), pl.ds(0, _L)], ix_all, xs).wait()

        def rowbase(t):
            g8 = wid * supers_pw + t // 8
            s = (g8 // b_groups) * 8 + t % 8
            return s * n_b + (g8 % b_groups) * _L

        def start_g(t, b):
            pltpu.async_copy(scr.at[ix_all.at[t]], rows[b], gsem[b])

        def wait_g(b):
            pltpu.make_async_copy(
                scr.at[pl.ds(0, _L)], rows[b], gsem[b]).wait()

        def start_o(t, b):
            pltpu.async_copy(
                rows[b], out.at[pl.ds(rowbase(t), _L)], osem[b])

        def wait_o(b):
            pltpu.make_async_copy(
                out.at[pl.ds(0, _L)], rows[b], osem[b]).wait()

        start_g(0, 0)
        start_g(1, 1)
        wait_g(0)
        start_o(0, 0)

        def body(i, carry):
            t_a = 2 * i + 1
            wait_o(0)
            start_g(t_a + 1, 0)
            wait_g(1)
            start_o(t_a, 1)
            wait_o(1)
            start_g(t_a + 2, 1)
            wait_g(0)
            start_o(t_a + 1, 0)
            return carry

        lax.fori_loop(0, (tasks_pw - 2) // 2, body, 0)
        wait_g(1)
        start_o(tasks_pw - 1, 1)
        wait_o(0)
        wait_o(1)

    return k(scratch, ids_t)


def _unpack_out(out128):
    n_s, n_b, blk = 200, 4096, 2048

    def body(x_ref, o_ref):
        o_ref[...] = jnp.transpose(x_ref[...])[: _D].reshape(1, _D, blk)

    return pl.pallas_call(
        body,
        grid=(n_s, n_b // blk),
        in_specs=[pl.BlockSpec((blk, _L), lambda s, b: (s * (n_b // blk) + b, 0))],
        out_specs=pl.BlockSpec((1, _D, blk), lambda s, b: (s, 0, b)),
        out_shape=jax.ShapeDtypeStruct((n_s, _D, n_b), jnp.float32),
    )(out128)


def kernel(input_ids, embedding_table):
    table_t = jnp.transpose(embedding_table)
    ids_t = jnp.transpose(input_ids.astype(jnp.int32))
    scratch = _pack_table(table_t)
    out128 = _sc_gather(scratch, ids_t)
    out3 = _unpack_out(out128)
    return jnp.transpose(out3, (2, 0, 1))


# speedup vs baseline: 2.5258x; 1.2330x over previous
import functools

import jax
import jax.numpy as jnp
from jax import lax
from jax.experimental import pallas as pl
from jax.experimental.pallas import tpu as pltpu
from jax.experimental.pallas import tpu_sc as plsc

_D = 64
_L = 128
_V = 1000000
_NW = 32


def _pack_table(table_t):
    blk = 16384
    grid = -(-_V // blk)

    def body(x_ref, o_ref):
        o_ref[:, : _D] = jnp.transpose(x_ref[...])

    return pl.pallas_call(
        body,
        grid=(grid,),
        in_specs=[pl.BlockSpec((_D, blk), lambda g: (0, g))],
        out_specs=pl.BlockSpec((blk, _L), lambda g: (g, 0)),
        out_shape=jax.ShapeDtypeStruct((_V, _L), jnp.float32),
    )(table_t)


def _sc_gather(scratch, ids_t):
    n_s, n_b = ids_t.shape
    b_groups = n_b // _L
    tasks_pw = n_s * b_groups // _NW
    supers_pw = tasks_pw // 8

    mesh = plsc.VectorSubcoreMesh(core_axis_name="c", subcore_axis_name="s")

    @functools.partial(
        pl.kernel,
        out_type=jax.ShapeDtypeStruct((n_s * n_b, _L), jnp.float32),
        mesh=mesh,
        scratch_types=[
            pltpu.VMEM((tasks_pw, _L), jnp.int32),
            pltpu.VMEM((_L, _L), jnp.float32),
            pltpu.VMEM((_L, _L), jnp.float32),
            pltpu.SemaphoreType.DMA,
            pltpu.SemaphoreType.DMA,
            pltpu.SemaphoreType.DMA,
            pltpu.SemaphoreType.DMA,
            pltpu.SemaphoreType.DMA,
        ],
        compiler_params=pltpu.CompilerParams(
            use_tc_tiling_on_sc=True, needs_layout_passes=False),
    )
    def k(scr, ids, out, ix_all, r0, r1, g0, g1, xs, o0, o1):
        wid = lax.axis_index("s") * 2 + lax.axis_index("c")
        rows = (r0, r1)
        gsem = (g0, g1)
        osem = (o0, o1)

        def stage(j, carry):
            g8 = wid * supers_pw + j
            s8 = g8 // b_groups
            b128 = g8 % b_groups
            pltpu.async_copy(
                ids.at[pl.ds(s8 * 8, 8), pl.ds(b128 * _L, _L)],
                ix_all.at[pl.ds(j * 8, 8)], xs)
            return carry

        lax.fori_loop(0, supers_pw, stage, 0)
        pltpu.make_async_copy(
            ids.at[pl.ds(0, tasks_pw), pl.ds(0, _L)], ix_all, xs).wait()

        def rowbase(t):
            g8 = wid * supers_pw + t // 8
            s = (g8 // b_groups) * 8 + t % 8
            return s * n_b + (g8 % b_groups) * _L

        def start_g(t, b):
            pltpu.async_copy(scr.at[ix_all.at[t]], rows[b], gsem[b])

        def wait_g(b):
            pltpu.make_async_copy(
                scr.at[pl.ds(0, _L)], rows[b], gsem[b]).wait()

        def start_o(t, b):
            pltpu.async_copy(
                rows[b], out.at[pl.ds(rowbase(t), _L)], osem[b])

        def wait_o(b):
            pltpu.make_async_copy(
                out.at[pl.ds(0, _L)], rows[b], osem[b]).wait()

        start_g(0, 0)
        start_g(1, 1)
        wait_g(0)
        start_o(0, 0)

        def body(i, carry):
            t_a = 2 * i + 1
            wait_o(0)
            start_g(t_a + 1, 0)
            wait_g(1)
            start_o(t_a, 1)
            wait_o(1)
            start_g(t_a + 2, 1)
            wait_g(0)
            start_o(t_a + 1, 0)
            return carry

        lax.fori_loop(0, (tasks_pw - 2) // 2, body, 0)
        wait_g(1)
        start_o(tasks_pw - 1, 1)
        wait_o(0)
        wait_o(1)

    return k(scratch, ids_t)


def _unpack_out(out128):
    n_s, n_b, blk = 200, 4096, 4096

    def body(x_ref, o_ref):
        o_ref[...] = jnp.transpose(x_ref[...])[: _D].reshape(1, _D, blk)

    return pl.pallas_call(
        body,
        grid=(n_s, n_b // blk),
        in_specs=[pl.BlockSpec((blk, _L), lambda s, b: (s * (n_b // blk) + b, 0))],
        out_specs=pl.BlockSpec((1, _D, blk), lambda s, b: (s, 0, b)),
        out_shape=jax.ShapeDtypeStruct((n_s, _D, n_b), jnp.float32),
    )(out128)


def kernel(input_ids, embedding_table):
    table_t = jnp.transpose(embedding_table)
    ids_t = jnp.transpose(input_ids.astype(jnp.int32))
    scratch = _pack_table(table_t)
    out128 = _sc_gather(scratch, ids_t)
    out3 = _unpack_out(out128)
    return jnp.transpose(out3, (2, 0, 1))
